# baseline (speedup 1.0000x reference)
import jax
import jax.numpy as jnp
from jax.experimental import pallas as pl

N_CAND = 7


def _relu6_body(x_ref, o_ref):
    o_ref[...] = jnp.minimum(jnp.maximum(x_ref[...], 0.0), 6.0)


def _relu6(x):
    return pl.pallas_call(
        _relu6_body,
        out_shape=jax.ShapeDtypeStruct(x.shape, x.dtype),
    )(x)


def kernel(x, edge_index, cand, W1, b1, W2, b2, W3, b3, Ws1, bs1, Ws2, bs2, We1, be1, We2, be2):
    feats = jnp.concatenate([x, cand], axis=0)
    Nt = feats.shape[0]
    loops = jnp.arange(Nt)
    src = jnp.concatenate([edge_index[0], loops])
    dst = jnp.concatenate([edge_index[1], loops])
    deg = jnp.zeros((Nt,), jnp.float32).at[dst].add(1.0)
    dinv = jnp.where(deg > 0, 1.0 / jnp.sqrt(deg), 0.0)
    norm = (dinv[src] * dinv[dst])[:, None]

    def gcn(h, W, b):
        hw = h @ W
        agg = jnp.zeros((Nt, W.shape[1]), jnp.float32).at[dst].add(hw[src] * norm)
        return agg + b

    h = feats
    for W, b in ((W1, b1), (W2, b2), (W3, b3)):
        h = _relu6(gcn(h, W, b))
    start_logits = _relu6((h @ Ws1 + bs1) @ Ws2 + bs2)
    idx = jnp.arange(Nt)
    s_masked = jnp.where(idx < Nt - N_CAND, start_logits[:, 0], -jnp.inf)
    start_node = jnp.argmax(s_masked)
    end_logits = _relu6((h @ We1 + be1) @ We2 + be2)
    e_masked = jnp.where(idx != start_node, end_logits[:, 0], -jnp.inf)
    end_node = jnp.argmax(e_masked)
    start_oh = jax.nn.one_hot(start_node, Nt, dtype=jnp.float32)
    end_oh = jax.nn.one_hot(end_node, Nt, dtype=jnp.float32)
    return (start_logits.squeeze(), start_oh, end_logits.squeeze(), end_oh)


# speedup vs baseline: 41.0009x; 41.0009x over previous
import functools

import jax
import jax.numpy as jnp
from jax import lax
from jax.experimental import pallas as pl
from jax.experimental.pallas import tpu as pltpu
from jax.experimental.pallas import tpu_sc as plsc

_N = 50007
_NREAL = 50000
_NPAD = 51200
_E = 1600000
_NC = 2
_NS = 16
_NW = _NC * _NS
_BPW = 392
_EPAD = _NW * _BPW * 128
_CHB = 8
_NCH = _BPW // _CHB
_RPS = _NPAD // _NS

_NB = 16
_RB = _NPAD // _NB

_f32 = jnp.float32


def _sc_mesh():
    return plsc.VectorSubcoreMesh(core_axis_name="c", subcore_axis_name="s")


def _make_deg_kernel():
    @functools.partial(
        pl.kernel,
        out_type=jax.ShapeDtypeStruct((_NC * _NPAD,), _f32),
        mesh=_sc_mesh(),
        compiler_params=pltpu.CompilerParams(use_tc_tiling_on_sc=False),
        scratch_types=[
            pltpu.VMEM((_CHB, 128), jnp.int32),
            pltpu.VMEM((128,), _f32),
            pltpu.VMEM_SHARED((_NPAD,), _f32),
        ],
    )
    def deg_kernel(dstb, zeros1, out, idx_d, ones_v, acc):
        c = lax.axis_index("c")
        s = lax.axis_index("s")
        w = c * _NS + s
        for i in range(8):
            ones_v[pl.ds(i * 16, 16)] = jnp.ones((16,), _f32)
        pltpu.sync_copy(zeros1.at[pl.ds(s * _RPS, _RPS)], acc.at[pl.ds(s * _RPS, _RPS)])
        plsc.subcore_barrier()

        def body(ch):
            pltpu.sync_copy(dstb.at[w, pl.ds(ch * _CHB, _CHB), :], idx_d)
            for j in range(_CHB):
                pltpu.sync_copy(ones_v, acc.at[idx_d.at[j]], add=True)

        pl.loop(0, _NCH)(body)
        plsc.subcore_barrier()
        pltpu.sync_copy(acc.at[pl.ds(s * _RPS, _RPS)], out.at[pl.ds(c * _NPAD + s * _RPS, _RPS)])

    return deg_kernel


def _make_scatter_kernel(D):
    @functools.partial(
        pl.kernel,
        out_type=jax.ShapeDtypeStruct((_NC, _NPAD, D), _f32),
        mesh=_sc_mesh(),
        compiler_params=pltpu.CompilerParams(use_tc_tiling_on_sc=False),
        scratch_types=[
            pltpu.VMEM((_CHB, 128), jnp.int32),
            pltpu.VMEM((_CHB, 128), jnp.int32),
            pltpu.VMEM((_CHB * 128, D), _f32),
            pltpu.VMEM_SHARED((_NPAD, D), _f32),
            pltpu.SemaphoreType.DMA,
        ],
    )
    def scat_kernel(hws, srcb, dstb, zeros, out, idx_s, idx_d, rows, acc, sem):
        c = lax.axis_index("c")
        s = lax.axis_index("s")
        w = c * _NS + s
        pltpu.sync_copy(zeros.at[pl.ds(s * _RPS, _RPS), :], acc.at[pl.ds(s * _RPS, _RPS), :])
        plsc.subcore_barrier()

        def body(ch):
            base = ch * _CHB
            pltpu.sync_copy(srcb.at[w, pl.ds(base, _CHB), :], idx_s)
            pltpu.sync_copy(dstb.at[w, pl.ds(base, _CHB), :], idx_d)
            handles = [
                pltpu.async_copy(
                    hws.at[idx_s.at[j]], rows.at[pl.ds(j * 128, 128), :], sem
                )
                for j in range(_CHB)
            ]
            for h in handles:
                h.wait()
            for j in range(_CHB):
                pltpu.sync_copy(
                    rows.at[pl.ds(j * 128, 128), :], acc.at[idx_d.at[j]], add=True
                )

        pl.loop(0, _NCH)(body)
        plsc.subcore_barrier()
        pltpu.sync_copy(
            acc.at[pl.ds(s * _RPS, _RPS), :], out.at[c, pl.ds(s * _RPS, _RPS), :]
        )

    return scat_kernel


def _tc1_body(p0, p1, feats, W1p, dinv, hws1):
    deg = 1.0 + p0[...] + p1[...]
    di = lax.rsqrt(deg)
    dinv[...] = di
    hws1[...] = di * jnp.dot(feats[...], W1p[...], preferred_element_type=_f32)


def _tc2_body(p0, p1, hws, dinv, b, Wn, out):
    di = dinv[...]
    h = jnp.clip(di * (p0[...] + p1[...] + hws[...]) + b[...], 0.0, 6.0)
    out[...] = di * jnp.dot(h, Wn[...], preferred_element_type=_f32)


def _tc2s_body(p0, p1, hws, dinv, b, Wn, outa, outb):
    di = dinv[...]
    h = jnp.clip(di * (p0[...] + p1[...] + hws[...]) + b[...], 0.0, 6.0)
    hw = di * jnp.dot(h, Wn[...], preferred_element_type=_f32)
    outa[...] = hw[:, :16]
    outb[...] = hw[:, 16:]


def _tc3_body(pa0, pa1, pb0, pb1, hwsa, hwsb, dinv, b3, Ws1, bs1, Ws2, bs2, We1, be1, We2, be2, sl, el):
    di = dinv[...]
    agg = jnp.concatenate(
        [pa0[...] + pa1[...] + hwsa[...], pb0[...] + pb1[...] + hwsb[...]], axis=1
    )
    h = jnp.clip(di * agg + b3[...], 0.0, 6.0)
    t = jnp.dot(h, Ws1[...], preferred_element_type=_f32) + bs1[...]
    sl[...] = jnp.clip(jnp.dot(t, Ws2[...], preferred_element_type=_f32) + bs2[...], 0.0, 6.0)
    u = jnp.dot(h, We1[...], preferred_element_type=_f32) + be1[...]
    el[...] = jnp.clip(jnp.dot(u, We2[...], preferred_element_type=_f32) + be2[...], 0.0, 6.0)


def _tc4_body(sl, el, soh, eoh):
    shp = (_NPAD // 128, 128)
    r = lax.broadcasted_iota(jnp.int32, shp, 0)
    col = lax.broadcasted_iota(jnp.int32, shp, 1)
    idx = r * 128 + col
    big = jnp.int32(2 ** 30)
    s = sl[...]
    sm = jnp.where(idx < _NREAL, s, -1.0)
    ms = jnp.max(sm)
    sidx = jnp.min(jnp.where(sm >= ms, idx, big))
    soh[...] = jnp.where(idx == sidx, 1.0, 0.0).astype(_f32)
    e = el[...]
    em = jnp.where(jnp.logical_and(idx != sidx, idx < _N), e, -1.0)
    me = jnp.max(em)
    eidx = jnp.min(jnp.where(em >= me, idx, big))
    eoh[...] = jnp.where(idx == eidx, 1.0, 0.0).astype(_f32)


def _row_spec(width):
    return pl.BlockSpec((_RB, width), lambda i: (i, 0))


def _full_spec(shape):
    return pl.BlockSpec(shape, lambda i: tuple(0 for _ in shape))


def _tc1(p0, p1, featsp, W1p):
    return pl.pallas_call(
        _tc1_body,
        grid=(_NB,),
        in_specs=[_row_spec(1), _row_spec(1), _row_spec(8), _full_spec((8, 16))],
        out_specs=[_row_spec(1), _row_spec(16)],
        out_shape=[
            jax.ShapeDtypeStruct((_NPAD, 1), _f32),
            jax.ShapeDtypeStruct((_NPAD, 16), _f32),
        ],
    )(p0, p1, featsp, W1p)


def _tc2(p0, p1, hws, dinv, b, Wn):
    Din = hws.shape[1]
    Dout = Wn.shape[1]
    return pl.pallas_call(
        _tc2_body,
        grid=(_NB,),
        in_specs=[
            _row_spec(Din), _row_spec(Din), _row_spec(Din), _row_spec(1),
            _full_spec((1, Din)), _full_spec((Din, Dout)),
        ],
        out_specs=_row_spec(Dout),
        out_shape=jax.ShapeDtypeStruct((_NPAD, Dout), _f32),
    )(p0, p1, hws, dinv, b, Wn)


def _tc2s(p0, p1, hws, dinv, b, Wn):
    Din = hws.shape[1]
    return pl.pallas_call(
        _tc2s_body,
        grid=(_NB,),
        in_specs=[
            _row_spec(Din), _row_spec(Din), _row_spec(Din), _row_spec(1),
            _full_spec((1, Din)), _full_spec((Din, 32)),
        ],
        out_specs=[_row_spec(16), _row_spec(16)],
        out_shape=[
            jax.ShapeDtypeStruct((_NPAD, 16), _f32),
            jax.ShapeDtypeStruct((_NPAD, 16), _f32),
        ],
    )(p0, p1, hws, dinv, b, Wn)


def _tc3(pa0, pa1, pb0, pb1, hwsa, hwsb, dinv, b3, Ws1, bs1, Ws2, bs2, We1, be1, We2, be2):
    return pl.pallas_call(
        _tc3_body,
        grid=(_NB,),
        in_specs=[
            _row_spec(16), _row_spec(16), _row_spec(16), _row_spec(16),
            _row_spec(16), _row_spec(16), _row_spec(1),
            _full_spec((1, 32)),
            _full_spec((32, 16)), _full_spec((1, 16)),
            _full_spec((16, 1)), _full_spec((1, 1)),
            _full_spec((32, 24)), _full_spec((1, 24)),
            _full_spec((24, 1)), _full_spec((1, 1)),
        ],
        out_specs=[_row_spec(1), _row_spec(1)],
        out_shape=[
            jax.ShapeDtypeStruct((_NPAD, 1), _f32),
            jax.ShapeDtypeStruct((_NPAD, 1), _f32),
        ],
    )(pa0, pa1, pb0, pb1, hwsa, hwsb, dinv, b3, Ws1, bs1, Ws2, bs2, We1, be1, We2, be2)


def _tc4(sl2d, el2d):
    return pl.pallas_call(
        _tc4_body,
        out_shape=[
            jax.ShapeDtypeStruct((_NPAD // 128, 128), _f32),
            jax.ShapeDtypeStruct((_NPAD // 128, 128), _f32),
        ],
    )(sl2d, el2d)


def kernel(x, edge_index, cand, W1, b1, W2, b2, W3, b3, Ws1, bs1, Ws2, bs2, We1, be1, We2, be2):
    feats = jnp.concatenate([x, cand], axis=0)
    featsp = jnp.zeros((_NPAD, 8), _f32).at[:_N, :7].set(feats)
    W1p = jnp.zeros((8, 16), _f32).at[:7].set(W1)

    src = edge_index[0].astype(jnp.int32)
    dst = edge_index[1].astype(jnp.int32)
    padn = _EPAD - _E
    pad_idx = _N + (jnp.arange(padn, dtype=jnp.int32) % 128)
    srcb = jnp.concatenate([src, pad_idx]).reshape(_NW, _BPW, 128)
    dstb = jnp.concatenate([dst, pad_idx]).reshape(_NW, _BPW, 128)

    zeros1 = jnp.zeros((_NPAD,), _f32)
    z16 = jnp.zeros((_NPAD, 16), _f32)
    z24 = jnp.zeros((_NPAD, 24), _f32)

    degp = _make_deg_kernel()(dstb, zeros1).reshape(_NC, _NPAD)
    p0 = degp[0][:, None]
    p1 = degp[1][:, None]

    dinv, hws1 = _tc1(p0, p1, featsp, W1p)

    s1 = _make_scatter_kernel(16)(hws1, srcb, dstb, z16)
    hws2 = _tc2(s1[0], s1[1], hws1, dinv, b1.reshape(1, -1), W2)
    s2 = _make_scatter_kernel(24)(hws2, srcb, dstb, z24)
    hws3a, hws3b = _tc2s(s2[0], s2[1], hws2, dinv, b2.reshape(1, -1), W3)
    sk16 = _make_scatter_kernel(16)
    s3a = sk16(hws3a, srcb, dstb, z16)
    s3b = sk16(hws3b, srcb, dstb, z16)

    sl, el = _tc3(
        s3a[0], s3a[1], s3b[0], s3b[1], hws3a, hws3b, dinv, b3.reshape(1, -1),
        Ws1, bs1.reshape(1, -1), Ws2, bs2.reshape(1, -1),
        We1, be1.reshape(1, -1), We2, be2.reshape(1, -1),
    )

    soh, eoh = _tc4(sl.reshape(_NPAD // 128, 128), el.reshape(_NPAD // 128, 128))

    start_logits = sl.reshape(-1)[:_N]
    end_logits = el.reshape(-1)[:_N]
    start_oh = soh.reshape(-1)[:_N]
    end_oh = eoh.reshape(-1)[:_N]
    return (start_logits, start_oh, end_logits, end_oh)


# speedup vs baseline: 54.3596x; 1.3258x over previous
import functools

import jax
import jax.numpy as jnp
from jax import lax
from jax.experimental import pallas as pl
from jax.experimental.pallas import tpu as pltpu
from jax.experimental.pallas import tpu_sc as plsc

_N = 50007
_NREAL = 50000
_NPAD = 51200
_E = 1600000
_NC = 2
_NS = 16
_NW = _NC * _NS
_BPW = 392
_EPAD = _NW * _BPW * 128
_CHB = 8
_NCH = _BPW // _CHB
_RPS = _NPAD // _NS

_NB = 16
_RB = _NPAD // _NB

_f32 = jnp.float32


def _sc_mesh():
    return plsc.VectorSubcoreMesh(core_axis_name="c", subcore_axis_name="s")


def _make_deg_kernel():
    @functools.partial(
        pl.kernel,
        out_type=jax.ShapeDtypeStruct((_NC * _NPAD,), _f32),
        mesh=_sc_mesh(),
        compiler_params=pltpu.CompilerParams(use_tc_tiling_on_sc=False),
        scratch_types=[
            pltpu.VMEM((_CHB, 128), jnp.int32),
            pltpu.VMEM((128,), _f32),
            pltpu.VMEM_SHARED((_NPAD,), _f32),
        ],
    )
    def deg_kernel(dstb, zeros1, out, idx_d, ones_v, acc):
        c = lax.axis_index("c")
        s = lax.axis_index("s")
        w = c * _NS + s
        for i in range(8):
            ones_v[pl.ds(i * 16, 16)] = jnp.ones((16,), _f32)
        pltpu.sync_copy(zeros1.at[pl.ds(s * _RPS, _RPS)], acc.at[pl.ds(s * _RPS, _RPS)])
        plsc.subcore_barrier()

        def body(ch):
            pltpu.sync_copy(dstb.at[w, pl.ds(ch * _CHB, _CHB), :], idx_d)
            for j in range(_CHB):
                pltpu.sync_copy(ones_v, acc.at[idx_d.at[j]], add=True)

        pl.loop(0, _NCH)(body)
        plsc.subcore_barrier()
        pltpu.sync_copy(acc.at[pl.ds(s * _RPS, _RPS)], out.at[pl.ds(c * _NPAD + s * _RPS, _RPS)])

    return deg_kernel


def _make_scatter_kernel(D):
    @functools.partial(
        pl.kernel,
        out_type=jax.ShapeDtypeStruct((_NC, _NPAD, D), _f32),
        mesh=_sc_mesh(),
        compiler_params=pltpu.CompilerParams(use_tc_tiling_on_sc=False),
        scratch_types=[
            pltpu.VMEM((_CHB, 128), jnp.int32),
            pltpu.VMEM((_CHB, 128), jnp.int32),
            pltpu.VMEM((_CHB, 128), jnp.int32),
            pltpu.VMEM((_CHB, 128), jnp.int32),
            pltpu.VMEM((_CHB * 128, D), _f32),
            pltpu.VMEM_SHARED((_NPAD, D), _f32),
            pltpu.SemaphoreType.DMA,
            pltpu.SemaphoreType.DMA,
            pltpu.SemaphoreType.DMA,
            pltpu.SemaphoreType.DMA,
        ],
    )
    def scat_kernel(hws, srcb, dstb, zeros, out,
                    idx_s0, idx_d0, idx_s1, idx_d1, rows, acc,
                    sem_i0, sem_i1, sem_g, sem_a):
        c = lax.axis_index("c")
        s = lax.axis_index("s")
        w = c * _NS + s
        pltpu.sync_copy(zeros.at[pl.ds(s * _RPS, _RPS), :], acc.at[pl.ds(s * _RPS, _RPS), :])
        plsc.subcore_barrier()

        bufs = ((idx_s0, idx_d0, sem_i0), (idx_s1, idx_d1, sem_i1))

        def issue_idx(ch, bset):
            isb, idb, sem = bset
            base = ch * _CHB
            pltpu.async_copy(srcb.at[w, pl.ds(base, _CHB), :], isb, sem)
            pltpu.async_copy(dstb.at[w, pl.ds(base, _CHB), :], idb, sem)

        def wait_idx(bset):
            isb, idb, sem = bset
            pltpu.make_async_copy(srcb.at[w, pl.ds(0, _CHB), :], isb, sem).wait()
            pltpu.make_async_copy(dstb.at[w, pl.ds(0, _CHB), :], idb, sem).wait()

        def do_chunk(ch, cur, nxt, guard_next):
            wait_idx(cur)
            if guard_next:
                @pl.when(ch + 1 < _NCH)
                def _():
                    issue_idx(ch + 1, nxt)
            else:
                issue_idx(ch + 1, nxt)
            isb, idb, _ = cur
            ghs = [
                pltpu.async_copy(hws.at[isb.at[j]], rows.at[pl.ds(j * 128, 128), :], sem_g)
                for j in range(_CHB)
            ]
            shs = []
            for j in range(_CHB):
                ghs[j].wait()
                shs.append(
                    pltpu.async_copy(
                        rows.at[pl.ds(j * 128, 128), :], acc.at[idb.at[j]], sem_a, add=True
                    )
                )
            for h in shs:
                h.wait()

        issue_idx(0, bufs[0])
        do_chunk(0, bufs[0], bufs[1], guard_next=False)

        def body(k):
            do_chunk(2 * k + 1, bufs[1], bufs[0], guard_next=False)
            do_chunk(2 * k + 2, bufs[0], bufs[1], guard_next=True)

        pl.loop(0, (_NCH - 1) // 2)(body)
        plsc.subcore_barrier()
        pltpu.sync_copy(
            acc.at[pl.ds(s * _RPS, _RPS), :], out.at[c, pl.ds(s * _RPS, _RPS), :]
        )

    return scat_kernel


def _tc1_body(p0, p1, feats, W1p, dinv, hws1):
    deg = 1.0 + p0[...] + p1[...]
    di = lax.rsqrt(deg)
    dinv[...] = di
    hws1[...] = di * jnp.dot(feats[...], W1p[...], preferred_element_type=_f32)


def _tc2_body(p0, p1, hws, dinv, b, Wn, out):
    di = dinv[...]
    h = jnp.clip(di * (p0[...] + p1[...] + hws[...]) + b[...], 0.0, 6.0)
    out[...] = di * jnp.dot(h, Wn[...], preferred_element_type=_f32)


def _tc2s_body(p0, p1, hws, dinv, b, Wn, outa, outb):
    di = dinv[...]
    h = jnp.clip(di * (p0[...] + p1[...] + hws[...]) + b[...], 0.0, 6.0)
    hw = di * jnp.dot(h, Wn[...], preferred_element_type=_f32)
    outa[...] = hw[:, :16]
    outb[...] = hw[:, 16:]


def _tc3_body(pa0, pa1, pb0, pb1, hwsa, hwsb, dinv, b3, Ws1, bs1, Ws2, bs2, We1, be1, We2, be2, sl, el):
    di = dinv[...]
    agg = jnp.concatenate(
        [pa0[...] + pa1[...] + hwsa[...], pb0[...] + pb1[...] + hwsb[...]], axis=1
    )
    h = jnp.clip(di * agg + b3[...], 0.0, 6.0)
    t = jnp.dot(h, Ws1[...], preferred_element_type=_f32) + bs1[...]
    sl[...] = jnp.clip(jnp.dot(t, Ws2[...], preferred_element_type=_f32) + bs2[...], 0.0, 6.0)
    u = jnp.dot(h, We1[...], preferred_element_type=_f32) + be1[...]
    el[...] = jnp.clip(jnp.dot(u, We2[...], preferred_element_type=_f32) + be2[...], 0.0, 6.0)


def _tc4_body(sl, el, soh, eoh):
    shp = (_NPAD // 128, 128)
    r = lax.broadcasted_iota(jnp.int32, shp, 0)
    col = lax.broadcasted_iota(jnp.int32, shp, 1)
    idx = r * 128 + col
    big = jnp.int32(2 ** 30)
    s = sl[...]
    sm = jnp.where(idx < _NREAL, s, -1.0)
    ms = jnp.max(sm)
    sidx = jnp.min(jnp.where(sm >= ms, idx, big))
    soh[...] = jnp.where(idx == sidx, 1.0, 0.0).astype(_f32)
    e = el[...]
    em = jnp.where(jnp.logical_and(idx != sidx, idx < _N), e, -1.0)
    me = jnp.max(em)
    eidx = jnp.min(jnp.where(em >= me, idx, big))
    eoh[...] = jnp.where(idx == eidx, 1.0, 0.0).astype(_f32)


def _row_spec(width):
    return pl.BlockSpec((_RB, width), lambda i: (i, 0))


def _full_spec(shape):
    return pl.BlockSpec(shape, lambda i: tuple(0 for _ in shape))


def _tc1(p0, p1, featsp, W1p):
    return pl.pallas_call(
        _tc1_body,
        grid=(_NB,),
        in_specs=[_row_spec(1), _row_spec(1), _row_spec(8), _full_spec((8, 16))],
        out_specs=[_row_spec(1), _row_spec(16)],
        out_shape=[
            jax.ShapeDtypeStruct((_NPAD, 1), _f32),
            jax.ShapeDtypeStruct((_NPAD, 16), _f32),
        ],
    )(p0, p1, featsp, W1p)


def _tc2(p0, p1, hws, dinv, b, Wn):
    Din = hws.shape[1]
    Dout = Wn.shape[1]
    return pl.pallas_call(
        _tc2_body,
        grid=(_NB,),
        in_specs=[
            _row_spec(Din), _row_spec(Din), _row_spec(Din), _row_spec(1),
            _full_spec((1, Din)), _full_spec((Din, Dout)),
        ],
        out_specs=_row_spec(Dout),
        out_shape=jax.ShapeDtypeStruct((_NPAD, Dout), _f32),
    )(p0, p1, hws, dinv, b, Wn)


def _tc2s(p0, p1, hws, dinv, b, Wn):
    Din = hws.shape[1]
    return pl.pallas_call(
        _tc2s_body,
        grid=(_NB,),
        in_specs=[
            _row_spec(Din), _row_spec(Din), _row_spec(Din), _row_spec(1),
            _full_spec((1, Din)), _full_spec((Din, 32)),
        ],
        out_specs=[_row_spec(16), _row_spec(16)],
        out_shape=[
            jax.ShapeDtypeStruct((_NPAD, 16), _f32),
            jax.ShapeDtypeStruct((_NPAD, 16), _f32),
        ],
    )(p0, p1, hws, dinv, b, Wn)


def _tc3(pa0, pa1, pb0, pb1, hwsa, hwsb, dinv, b3, Ws1, bs1, Ws2, bs2, We1, be1, We2, be2):
    return pl.pallas_call(
        _tc3_body,
        grid=(_NB,),
        in_specs=[
            _row_spec(16), _row_spec(16), _row_spec(16), _row_spec(16),
            _row_spec(16), _row_spec(16), _row_spec(1),
            _full_spec((1, 32)),
            _full_spec((32, 16)), _full_spec((1, 16)),
            _full_spec((16, 1)), _full_spec((1, 1)),
            _full_spec((32, 24)), _full_spec((1, 24)),
            _full_spec((24, 1)), _full_spec((1, 1)),
        ],
        out_specs=[_row_spec(1), _row_spec(1)],
        out_shape=[
            jax.ShapeDtypeStruct((_NPAD, 1), _f32),
            jax.ShapeDtypeStruct((_NPAD, 1), _f32),
        ],
    )(pa0, pa1, pb0, pb1, hwsa, hwsb, dinv, b3, Ws1, bs1, Ws2, bs2, We1, be1, We2, be2)


def _tc4(sl2d, el2d):
    return pl.pallas_call(
        _tc4_body,
        out_shape=[
            jax.ShapeDtypeStruct((_NPAD // 128, 128), _f32),
            jax.ShapeDtypeStruct((_NPAD // 128, 128), _f32),
        ],
    )(sl2d, el2d)


def kernel(x, edge_index, cand, W1, b1, W2, b2, W3, b3, Ws1, bs1, Ws2, bs2, We1, be1, We2, be2):
    feats = jnp.concatenate([x, cand], axis=0)
    featsp = jnp.zeros((_NPAD, 8), _f32).at[:_N, :7].set(feats)
    W1p = jnp.zeros((8, 16), _f32).at[:7].set(W1)

    src = edge_index[0].astype(jnp.int32)
    dst = edge_index[1].astype(jnp.int32)
    padn = _EPAD - _E
    pad_idx = _N + (jnp.arange(padn, dtype=jnp.int32) % 128)
    srcb = jnp.concatenate([src, pad_idx]).reshape(_NW, _BPW, 128)
    dstb = jnp.concatenate([dst, pad_idx]).reshape(_NW, _BPW, 128)

    zeros1 = jnp.zeros((_NPAD,), _f32)
    z16 = jnp.zeros((_NPAD, 16), _f32)
    z24 = jnp.zeros((_NPAD, 24), _f32)

    degp = _make_deg_kernel()(dstb, zeros1).reshape(_NC, _NPAD)
    p0 = degp[0][:, None]
    p1 = degp[1][:, None]

    dinv, hws1 = _tc1(p0, p1, featsp, W1p)

    s1 = _make_scatter_kernel(16)(hws1, srcb, dstb, z16)
    hws2 = _tc2(s1[0], s1[1], hws1, dinv, b1.reshape(1, -1), W2)
    s2 = _make_scatter_kernel(24)(hws2, srcb, dstb, z24)
    hws3a, hws3b = _tc2s(s2[0], s2[1], hws2, dinv, b2.reshape(1, -1), W3)
    sk16 = _make_scatter_kernel(16)
    s3a = sk16(hws3a, srcb, dstb, z16)
    s3b = sk16(hws3b, srcb, dstb, z16)

    sl, el = _tc3(
        s3a[0], s3a[1], s3b[0], s3b[1], hws3a, hws3b, dinv, b3.reshape(1, -1),
        Ws1, bs1.reshape(1, -1), Ws2, bs2.reshape(1, -1),
        We1, be1.reshape(1, -1), We2, be2.reshape(1, -1),
    )

    soh, eoh = _tc4(sl.reshape(_NPAD // 128, 128), el.reshape(_NPAD // 128, 128))

    start_logits = sl.reshape(-1)[:_N]
    end_logits = el.reshape(-1)[:_N]
    start_oh = soh.reshape(-1)[:_N]
    end_oh = eoh.reshape(-1)[:_N]
    return (start_logits, start_oh, end_logits, end_oh)


# speedup vs baseline: 57.9763x; 1.0665x over previous
import functools

import jax
import jax.numpy as jnp
from jax import lax
from jax.experimental import pallas as pl
from jax.experimental.pallas import tpu as pltpu
from jax.experimental.pallas import tpu_sc as plsc

_N = 50007
_NREAL = 50000
_NPAD = 51200
_E = 1600000
_NC = 2
_NS = 16
_NW = _NC * _NS
_BPW = 392
_EPAD = _NW * _BPW * 128
_CHB = 8
_NCH = _BPW // _CHB
_BPW2 = _EPAD // (_NS * 128)
_NCH2 = _BPW2 // _CHB
_RPS = _NPAD // _NS

_NB = 16
_RB = _NPAD // _NB

_f32 = jnp.float32


def _sc_mesh():
    return plsc.VectorSubcoreMesh(core_axis_name="c", subcore_axis_name="s")


def _idx_ops(srcb, dstb, w):
    def issue_idx(ch, bset):
        isb, idb, sem = bset
        base = ch * _CHB
        pltpu.async_copy(srcb.at[w, pl.ds(base, _CHB), :], isb, sem)
        pltpu.async_copy(dstb.at[w, pl.ds(base, _CHB), :], idb, sem)

    def wait_idx(bset):
        isb, idb, sem = bset
        pltpu.make_async_copy(srcb.at[w, pl.ds(0, _CHB), :], isb, sem).wait()
        pltpu.make_async_copy(dstb.at[w, pl.ds(0, _CHB), :], idb, sem).wait()

    return issue_idx, wait_idx


def _sweep(table, srcb, dstb, w, nch, bufs, rows, acc, sem_g, sem_a):
    issue_idx, wait_idx = _idx_ops(srcb, dstb, w)

    def do_chunk(ch, cur, nxt, issue_next, guard_next):
        wait_idx(cur)
        if issue_next:
            if guard_next:
                @pl.when(ch + 1 < nch)
                def _():
                    issue_idx(ch + 1, nxt)
            else:
                issue_idx(ch + 1, nxt)
        isb, idb, _ = cur
        ghs = [
            pltpu.async_copy(table.at[isb.at[j]], rows.at[pl.ds(j * 128, 128), :], sem_g)
            for j in range(_CHB)
        ]
        shs = []
        for j in range(_CHB):
            ghs[j].wait()
            shs.append(
                pltpu.async_copy(
                    rows.at[pl.ds(j * 128, 128), :], acc.at[idb.at[j]], sem_a, add=True
                )
            )
        for h in shs:
            h.wait()

    issue_idx(0, bufs[0])
    do_chunk(0, bufs[0], bufs[1], issue_next=True, guard_next=False)
    npairs = (nch - 1) // 2

    def body(k):
        do_chunk(2 * k + 1, bufs[1], bufs[0], issue_next=True, guard_next=False)
        do_chunk(2 * k + 2, bufs[0], bufs[1], issue_next=True, guard_next=True)

    pl.loop(0, npairs)(body)
    if (nch - 1) % 2 == 1:
        last = nch - 1
        do_chunk(last, bufs[last % 2], bufs[1 - last % 2], issue_next=False, guard_next=False)


def _make_deg_kernel():
    @functools.partial(
        pl.kernel,
        out_type=jax.ShapeDtypeStruct((_NC * _NPAD,), _f32),
        mesh=_sc_mesh(),
        compiler_params=pltpu.CompilerParams(use_tc_tiling_on_sc=False),
        scratch_types=[
            pltpu.VMEM((_CHB, 128), jnp.int32),
            pltpu.VMEM((_CHB, 128), jnp.int32),
            pltpu.VMEM((128,), _f32),
            pltpu.VMEM_SHARED((_NPAD,), _f32),
            pltpu.SemaphoreType.DMA,
            pltpu.SemaphoreType.DMA,
            pltpu.SemaphoreType.DMA,
        ],
    )
    def deg_kernel(dstb, zeros1, out, idx_d0, idx_d1, ones_v, acc, sem_i0, sem_i1, sem_a):
        c = lax.axis_index("c")
        s = lax.axis_index("s")
        w = c * _NS + s
        for i in range(8):
            ones_v[pl.ds(i * 16, 16)] = jnp.ones((16,), _f32)
        pltpu.sync_copy(zeros1.at[pl.ds(s * _RPS, _RPS)], acc.at[pl.ds(s * _RPS, _RPS)])
        plsc.subcore_barrier()

        bufs = ((idx_d0, sem_i0), (idx_d1, sem_i1))

        def issue_idx(ch, bset):
            idb, sem = bset
            pltpu.async_copy(dstb.at[w, pl.ds(ch * _CHB, _CHB), :], idb, sem)

        def wait_idx(bset):
            idb, sem = bset
            pltpu.make_async_copy(dstb.at[w, pl.ds(0, _CHB), :], idb, sem).wait()

        def do_chunk(ch, cur, nxt, issue_next, guard_next):
            wait_idx(cur)
            if issue_next:
                if guard_next:
                    @pl.when(ch + 1 < _NCH)
                    def _():
                        issue_idx(ch + 1, nxt)
                else:
                    issue_idx(ch + 1, nxt)
            idb, _ = cur
            shs = [
                pltpu.async_copy(ones_v, acc.at[idb.at[j]], sem_a, add=True)
                for j in range(_CHB)
            ]
            for h in shs:
                h.wait()

        issue_idx(0, bufs[0])
        do_chunk(0, bufs[0], bufs[1], issue_next=True, guard_next=False)

        def body(k):
            do_chunk(2 * k + 1, bufs[1], bufs[0], issue_next=True, guard_next=False)
            do_chunk(2 * k + 2, bufs[0], bufs[1], issue_next=True, guard_next=True)

        pl.loop(0, (_NCH - 1) // 2)(body)
        plsc.subcore_barrier()
        pltpu.sync_copy(acc.at[pl.ds(s * _RPS, _RPS)], out.at[pl.ds(c * _NPAD + s * _RPS, _RPS)])

    return deg_kernel


def _make_scatter_kernel(D):
    @functools.partial(
        pl.kernel,
        out_type=jax.ShapeDtypeStruct((_NC, _NPAD, D), _f32),
        mesh=_sc_mesh(),
        compiler_params=pltpu.CompilerParams(use_tc_tiling_on_sc=False),
        scratch_types=[
            pltpu.VMEM((_CHB, 128), jnp.int32),
            pltpu.VMEM((_CHB, 128), jnp.int32),
            pltpu.VMEM((_CHB, 128), jnp.int32),
            pltpu.VMEM((_CHB, 128), jnp.int32),
            pltpu.VMEM((_CHB * 128, D), _f32),
            pltpu.VMEM_SHARED((_NPAD, D), _f32),
            pltpu.SemaphoreType.DMA,
            pltpu.SemaphoreType.DMA,
            pltpu.SemaphoreType.DMA,
            pltpu.SemaphoreType.DMA,
        ],
    )
    def scat_kernel(hws, srcb, dstb, zeros, out,
                    idx_s0, idx_d0, idx_s1, idx_d1, rows, acc,
                    sem_i0, sem_i1, sem_g, sem_a):
        c = lax.axis_index("c")
        s = lax.axis_index("s")
        w = c * _NS + s
        pltpu.sync_copy(zeros.at[pl.ds(s * _RPS, _RPS), :], acc.at[pl.ds(s * _RPS, _RPS), :])
        plsc.subcore_barrier()
        bufs = ((idx_s0, idx_d0, sem_i0), (idx_s1, idx_d1, sem_i1))
        _sweep(hws, srcb, dstb, w, _NCH, bufs, rows, acc, sem_g, sem_a)
        plsc.subcore_barrier()
        pltpu.sync_copy(
            acc.at[pl.ds(s * _RPS, _RPS), :], out.at[c, pl.ds(s * _RPS, _RPS), :]
        )

    return scat_kernel


def _make_scatter3_kernel():
    @functools.partial(
        pl.kernel,
        out_type=jax.ShapeDtypeStruct((_NC, _NPAD, 16), _f32),
        mesh=_sc_mesh(),
        compiler_params=pltpu.CompilerParams(use_tc_tiling_on_sc=False),
        scratch_types=[
            pltpu.VMEM((_CHB, 128), jnp.int32),
            pltpu.VMEM((_CHB, 128), jnp.int32),
            pltpu.VMEM((_CHB, 128), jnp.int32),
            pltpu.VMEM((_CHB, 128), jnp.int32),
            pltpu.VMEM((_CHB * 128, 16), _f32),
            pltpu.VMEM_SHARED((_NPAD, 16), _f32),
            pltpu.SemaphoreType.DMA,
            pltpu.SemaphoreType.DMA,
            pltpu.SemaphoreType.DMA,
            pltpu.SemaphoreType.DMA,
        ],
    )
    def scat3_kernel(hwsa, hwsb, srcb2, dstb2, zeros, out,
                     idx_s0, idx_d0, idx_s1, idx_d1, rows, acc,
                     sem_i0, sem_i1, sem_g, sem_a):
        c = lax.axis_index("c")
        s = lax.axis_index("s")
        pltpu.sync_copy(zeros.at[pl.ds(s * _RPS, _RPS), :], acc.at[pl.ds(s * _RPS, _RPS), :])
        plsc.subcore_barrier()
        bufs = ((idx_s0, idx_d0, sem_i0), (idx_s1, idx_d1, sem_i1))

        @pl.when(c == 0)
        def _():
            _sweep(hwsa, srcb2, dstb2, s, _NCH2, bufs, rows, acc, sem_g, sem_a)

        @pl.when(c == 1)
        def _():
            _sweep(hwsb, srcb2, dstb2, s, _NCH2, bufs, rows, acc, sem_g, sem_a)

        plsc.subcore_barrier()
        pltpu.sync_copy(
            acc.at[pl.ds(s * _RPS, _RPS), :], out.at[c, pl.ds(s * _RPS, _RPS), :]
        )

    return scat3_kernel


def _tc1_body(p0, p1, feats, W1p, dinv, hws1):
    deg = 1.0 + p0[...] + p1[...]
    di = lax.rsqrt(deg)
    dinv[...] = di
    hws1[...] = di * jnp.dot(feats[...], W1p[...], preferred_element_type=_f32)


def _tc2_body(p0, p1, hws, dinv, b, Wn, out):
    di = dinv[...]
    h = jnp.clip(di * (p0[...] + p1[...] + hws[...]) + b[...], 0.0, 6.0)
    out[...] = di * jnp.dot(h, Wn[...], preferred_element_type=_f32)


def _tc2s_body(p0, p1, hws, dinv, b, Wn, outa, outb):
    di = dinv[...]
    h = jnp.clip(di * (p0[...] + p1[...] + hws[...]) + b[...], 0.0, 6.0)
    hw = di * jnp.dot(h, Wn[...], preferred_element_type=_f32)
    outa[...] = hw[:, :16]
    outb[...] = hw[:, 16:]


def _tc3_body(pa, pb, hwsa, hwsb, dinv, b3, Ws1, bs1, Ws2, bs2, We1, be1, We2, be2, sl, el):
    di = dinv[...]
    agg = jnp.concatenate([pa[...] + hwsa[...], pb[...] + hwsb[...]], axis=1)
    h = jnp.clip(di * agg + b3[...], 0.0, 6.0)
    t = jnp.dot(h, Ws1[...], preferred_element_type=_f32) + bs1[...]
    sl[...] = jnp.clip(jnp.dot(t, Ws2[...], preferred_element_type=_f32) + bs2[...], 0.0, 6.0)
    u = jnp.dot(h, We1[...], preferred_element_type=_f32) + be1[...]
    el[...] = jnp.clip(jnp.dot(u, We2[...], preferred_element_type=_f32) + be2[...], 0.0, 6.0)


def _tc4_body(sl, el, soh, eoh):
    shp = (_NPAD // 128, 128)
    r = lax.broadcasted_iota(jnp.int32, shp, 0)
    col = lax.broadcasted_iota(jnp.int32, shp, 1)
    idx = r * 128 + col
    big = jnp.int32(2 ** 30)
    s = sl[...]
    sm = jnp.where(idx < _NREAL, s, -1.0)
    ms = jnp.max(sm)
    sidx = jnp.min(jnp.where(sm >= ms, idx, big))
    soh[...] = jnp.where(idx == sidx, 1.0, 0.0).astype(_f32)
    e = el[...]
    em = jnp.where(jnp.logical_and(idx != sidx, idx < _N), e, -1.0)
    me = jnp.max(em)
    eidx = jnp.min(jnp.where(em >= me, idx, big))
    eoh[...] = jnp.where(idx == eidx, 1.0, 0.0).astype(_f32)


def _row_spec(width):
    return pl.BlockSpec((_RB, width), lambda i: (i, 0))


def _full_spec(shape):
    return pl.BlockSpec(shape, lambda i: tuple(0 for _ in shape))


def _tc1(p0, p1, featsp, W1p):
    return pl.pallas_call(
        _tc1_body,
        grid=(_NB,),
        in_specs=[_row_spec(1), _row_spec(1), _row_spec(8), _full_spec((8, 16))],
        out_specs=[_row_spec(1), _row_spec(16)],
        out_shape=[
            jax.ShapeDtypeStruct((_NPAD, 1), _f32),
            jax.ShapeDtypeStruct((_NPAD, 16), _f32),
        ],
    )(p0, p1, featsp, W1p)


def _tc2(p0, p1, hws, dinv, b, Wn):
    Din = hws.shape[1]
    Dout = Wn.shape[1]
    return pl.pallas_call(
        _tc2_body,
        grid=(_NB,),
        in_specs=[
            _row_spec(Din), _row_spec(Din), _row_spec(Din), _row_spec(1),
            _full_spec((1, Din)), _full_spec((Din, Dout)),
        ],
        out_specs=_row_spec(Dout),
        out_shape=jax.ShapeDtypeStruct((_NPAD, Dout), _f32),
    )(p0, p1, hws, dinv, b, Wn)


def _tc2s(p0, p1, hws, dinv, b, Wn):
    Din = hws.shape[1]
    return pl.pallas_call(
        _tc2s_body,
        grid=(_NB,),
        in_specs=[
            _row_spec(Din), _row_spec(Din), _row_spec(Din), _row_spec(1),
            _full_spec((1, Din)), _full_spec((Din, 32)),
        ],
        out_specs=[_row_spec(16), _row_spec(16)],
        out_shape=[
            jax.ShapeDtypeStruct((_NPAD, 16), _f32),
            jax.ShapeDtypeStruct((_NPAD, 16), _f32),
        ],
    )(p0, p1, hws, dinv, b, Wn)


def _tc3(pa, pb, hwsa, hwsb, dinv, b3, Ws1, bs1, Ws2, bs2, We1, be1, We2, be2):
    return pl.pallas_call(
        _tc3_body,
        grid=(_NB,),
        in_specs=[
            _row_spec(16), _row_spec(16),
            _row_spec(16), _row_spec(16), _row_spec(1),
            _full_spec((1, 32)),
            _full_spec((32, 16)), _full_spec((1, 16)),
            _full_spec((16, 1)), _full_spec((1, 1)),
            _full_spec((32, 24)), _full_spec((1, 24)),
            _full_spec((24, 1)), _full_spec((1, 1)),
        ],
        out_specs=[_row_spec(1), _row_spec(1)],
        out_shape=[
            jax.ShapeDtypeStruct((_NPAD, 1), _f32),
            jax.ShapeDtypeStruct((_NPAD, 1), _f32),
        ],
    )(pa, pb, hwsa, hwsb, dinv, b3, Ws1, bs1, Ws2, bs2, We1, be1, We2, be2)


def _tc4(sl2d, el2d):
    return pl.pallas_call(
        _tc4_body,
        out_shape=[
            jax.ShapeDtypeStruct((_NPAD // 128, 128), _f32),
            jax.ShapeDtypeStruct((_NPAD // 128, 128), _f32),
        ],
    )(sl2d, el2d)


def kernel(x, edge_index, cand, W1, b1, W2, b2, W3, b3, Ws1, bs1, Ws2, bs2, We1, be1, We2, be2):
    feats = jnp.concatenate([x, cand], axis=0)
    featsp = jnp.zeros((_NPAD, 8), _f32).at[:_N, :7].set(feats)
    W1p = jnp.zeros((8, 16), _f32).at[:7].set(W1)

    src = edge_index[0].astype(jnp.int32)
    dst = edge_index[1].astype(jnp.int32)
    padn = _EPAD - _E
    pad_idx = _N + (jnp.arange(padn, dtype=jnp.int32) % 128)
    srcb = jnp.concatenate([src, pad_idx]).reshape(_NW, _BPW, 128)
    dstb = jnp.concatenate([dst, pad_idx]).reshape(_NW, _BPW, 128)

    zeros1 = jnp.zeros((_NPAD,), _f32)
    z16 = jnp.zeros((_NPAD, 16), _f32)
    z24 = jnp.zeros((_NPAD, 24), _f32)

    degp = _make_deg_kernel()(dstb, zeros1).reshape(_NC, _NPAD)
    p0 = degp[0][:, None]
    p1 = degp[1][:, None]

    dinv, hws1 = _tc1(p0, p1, featsp, W1p)

    s1 = _make_scatter_kernel(16)(hws1, srcb, dstb, z16)
    hws2 = _tc2(s1[0], s1[1], hws1, dinv, b1.reshape(1, -1), W2)
    s2 = _make_scatter_kernel(24)(hws2, srcb, dstb, z24)
    hws3a, hws3b = _tc2s(s2[0], s2[1], hws2, dinv, b2.reshape(1, -1), W3)
    srcb2 = srcb.reshape(_NS, _BPW2, 128)
    dstb2 = dstb.reshape(_NS, _BPW2, 128)
    s3 = _make_scatter3_kernel()(hws3a, hws3b, srcb2, dstb2, z16)

    sl, el = _tc3(
        s3[0], s3[1], hws3a, hws3b, dinv, b3.reshape(1, -1),
        Ws1, bs1.reshape(1, -1), Ws2, bs2.reshape(1, -1),
        We1, be1.reshape(1, -1), We2, be2.reshape(1, -1),
    )

    soh, eoh = _tc4(sl.reshape(_NPAD // 128, 128), el.reshape(_NPAD // 128, 128))

    start_logits = sl.reshape(-1)[:_N]
    end_logits = el.reshape(-1)[:_N]
    start_oh = soh.reshape(-1)[:_N]
    end_oh = eoh.reshape(-1)[:_N]
    return (start_logits, start_oh, end_logits, end_oh)
